# baseline (speedup 1.0000x reference)
import functools

import jax
import jax.numpy as jnp
from jax import lax
from jax.experimental import pallas as pl
from jax.experimental.pallas import tpu as pltpu
from jax.experimental.pallas import tpu_sc as plsc

BATCH = 16384
DIM = 64
L = 16
NC, NS = 2, 16
NW = NC * NS
CHUNK = BATCH // NW
NSUB = 2
SUB = CHUNK // NSUB
NG = SUB // L
NSEM = 8
RPS = SUB // NSEM

_mesh = plsc.VectorSubcoreMesh(core_axis_name="c", subcore_axis_name="s")


@functools.partial(
    pl.kernel,
    out_type=jax.ShapeDtypeStruct((BATCH,), jnp.float32),
    mesh=_mesh,
    compiler_params=pltpu.CompilerParams(
        needs_layout_passes=False, use_tc_tiling_on_sc=True),
    scratch_types=[
        pltpu.VMEM((CHUNK,), jnp.int32),
        pltpu.VMEM((SUB, DIM), jnp.float32),
        pltpu.VMEM((SUB, DIM), jnp.float32),
        pltpu.VMEM((SUB, DIM), jnp.float32),
        pltpu.VMEM((CHUNK,), jnp.float32),
        [pltpu.SemaphoreType.DMA] * NSEM,
        pltpu.SemaphoreType.DMA,
    ],
)
def _distmult_sc(d1_hbm, d2_hbm, ctx_hbm, rel_hbm, out_hbm,
                 idx_v, rel_v, d1_v, d2_v, out_v, gsems, dsem):
    wid = lax.axis_index("s") * NC + lax.axis_index("c")
    base = wid * CHUNK

    pltpu.sync_copy(ctx_hbm.at[pl.ds(base, CHUNK)], idx_v)

    for h in range(NSUB):
        hoff = h * SUB
        cp1 = pltpu.async_copy(
            d1_hbm.at[pl.ds(base + hoff, SUB)], d1_v, dsem)
        cp2 = pltpu.async_copy(
            d2_hbm.at[pl.ds(base + hoff, SUB)], d2_v, dsem)

        def issue(i, carry, hoff=hoff):
            iv = idx_v[pl.ds(hoff + i * L, L)]
            for j in range(L):
                pltpu.async_copy(
                    rel_hbm.at[iv[j]], rel_v.at[i * L + j],
                    gsems[j % NSEM])
            return carry

        lax.fori_loop(0, SUB // L, issue, 0)
        for k in range(NSEM):
            pltpu.make_async_copy(
                rel_hbm.at[pl.ds(0, RPS)],
                rel_v.at[pl.ds(k * RPS, RPS)], gsems[k]).wait()
        cp1.wait()
        cp2.wait()

        def group(g, carry, hoff=hoff):
            outv = jnp.zeros((L,), jnp.float32)
            for j in range(L):
                r = g * L + j
                acc = jnp.zeros((L,), jnp.float32)
                for c in range(DIM // L):
                    s = pl.ds(c * L, L)
                    acc += d1_v[r, s] * rel_v[r, s] * d2_v[r, s]
                lane = lax.iota(jnp.int32, L)
                outv = jnp.where(lane == j, jnp.sum(acc), outv)
            out_v[pl.ds(hoff + g * L, L)] = outv
            return carry

        lax.fori_loop(0, NG, group, 0)

    pltpu.sync_copy(out_v, out_hbm.at[pl.ds(base, CHUNK)])


def kernel(d1_embd, d2_embd, context_ids, drug_1_ids, drug_2_ids, relation):
    return _distmult_sc(
        d1_embd, d2_embd, context_ids.astype(jnp.int32), relation)


# speedup vs baseline: 1.0014x; 1.0014x over previous
import functools

import jax
import jax.numpy as jnp
from jax import lax
from jax.experimental import pallas as pl
from jax.experimental.pallas import tpu as pltpu
from jax.experimental.pallas import tpu_sc as plsc

BATCH = 16384
DIM = 64
L = 16
NC, NS = 2, 16
NW = NC * NS
CHUNK = BATCH // NW
NQ = 4
Q = CHUNK // NQ
NG = Q // L

_mesh = plsc.VectorSubcoreMesh(core_axis_name="c", subcore_axis_name="s")


@functools.partial(
    pl.kernel,
    out_type=jax.ShapeDtypeStruct((BATCH,), jnp.float32),
    mesh=_mesh,
    compiler_params=pltpu.CompilerParams(
        needs_layout_passes=False, use_tc_tiling_on_sc=True),
    scratch_types=[
        pltpu.VMEM((CHUNK,), jnp.int32),
        pltpu.VMEM((Q, DIM), jnp.float32),
        pltpu.VMEM((Q, DIM), jnp.float32),
        pltpu.VMEM((Q, DIM), jnp.float32),
        pltpu.VMEM((Q, DIM), jnp.float32),
        pltpu.VMEM((Q, DIM), jnp.float32),
        pltpu.VMEM((Q, DIM), jnp.float32),
        pltpu.VMEM((CHUNK,), jnp.float32),
        pltpu.SemaphoreType.DMA,
        pltpu.SemaphoreType.DMA,
        pltpu.SemaphoreType.DMA,
        pltpu.SemaphoreType.DMA,
    ],
)
def _distmult_sc(d1_hbm, d2_hbm, ctx_hbm, rel_hbm, out_hbm,
                 idx_v, rb0, rb1, d1b0, d1b1, d2b0, d2b1, out_v,
                 gsem0, gsem1, dsem0, dsem1):
    wid = lax.axis_index("s") * NC + lax.axis_index("c")
    base = wid * CHUNK
    rbs = (rb0, rb1)
    gsems = (gsem0, gsem1)
    d1bs = (d1b0, d1b1)
    d2bs = (d2b0, d2b1)
    dsems = (dsem0, dsem1)

    pltpu.sync_copy(ctx_hbm.at[pl.ds(base, CHUNK)], idx_v)

    def fire(q, buf):
        pltpu.async_copy(
            d1_hbm.at[pl.ds(base + q * Q, Q)], d1bs[buf], dsems[buf])
        pltpu.async_copy(
            d2_hbm.at[pl.ds(base + q * Q, Q)], d2bs[buf], dsems[buf])

        def issue(i, carry):
            iv = idx_v[pl.ds(q * Q + i * L, L)]
            for j in range(L):
                pltpu.async_copy(
                    rel_hbm.at[iv[j]], rbs[buf].at[i * L + j], gsems[buf])
            return carry

        lax.fori_loop(0, Q // L, issue, 0)

    def wait(buf):
        pltpu.make_async_copy(
            rel_hbm.at[pl.ds(0, Q)], rbs[buf], gsems[buf]).wait()
        pltpu.make_async_copy(
            d1_hbm.at[pl.ds(0, Q)], d1bs[buf], dsems[buf]).wait()
        pltpu.make_async_copy(
            d2_hbm.at[pl.ds(0, Q)], d2bs[buf], dsems[buf]).wait()

    def compute(q, buf):
        rb, d1b, d2b = rbs[buf], d1bs[buf], d2bs[buf]

        def group(g, carry):
            outv = jnp.zeros((L,), jnp.float32)
            for j in range(L):
                r = g * L + j
                acc = jnp.zeros((L,), jnp.float32)
                for c in range(DIM // L):
                    s = pl.ds(c * L, L)
                    acc += d1b[r, s] * rb[r, s] * d2b[r, s]
                lane = lax.iota(jnp.int32, L)
                outv = jnp.where(lane == j, jnp.sum(acc), outv)
            out_v[pl.ds(q * Q + g * L, L)] = outv
            return carry

        lax.fori_loop(0, NG, group, 0)

    fire(0, 0)
    for q in range(NQ):
        buf = q % 2
        if q + 1 < NQ:
            fire(q + 1, 1 - buf)
        wait(buf)
        compute(q, buf)

    pltpu.sync_copy(out_v, out_hbm.at[pl.ds(base, CHUNK)])


def kernel(d1_embd, d2_embd, context_ids, drug_1_ids, drug_2_ids, relation):
    return _distmult_sc(
        d1_embd, d2_embd, context_ids.astype(jnp.int32), relation)
